# baseline (speedup 1.0000x reference)
import functools

import jax
import jax.numpy as jnp
from jax import lax
from jax.experimental import pallas as pl
from jax.experimental.pallas import tpu as pltpu
from jax.experimental.pallas import tpu_sc as plsc

N = 10000
E = 160000
C = 128
M = 20000
K = 3

NCORE = 2
NSUB = 16
LANES = 16

CH = C // 4
NQ = 2

CE = 128
CPT = 80
EPAD = CE * CPT * NSUB
NACC = 10240
RPT = NACC // NSUB

QB = 400
GS = 8
NG = N // GS

PC = 128
GPT = 5
MPAD = PC * GPT * NCORE * NSUB


def _edge_agg_body(x4, posx, posy, srcp, dstp,
                   sx_out, sy_out, s1_out, scal_out,
                   sx_acc, sy_acc, s1_acc, scal_acc, posx_sh, posy_sh,
                   posx_v, posy_v, src_v, dst_v, idx2_v, dsti_v,
                   wx_v, wy_v, xs_v, wxb_v, wyb_v, scr_v,
                   stsem0, stsem1, gsem0, gsem1, ssem0, ssem1):
    half = lax.axis_index("c")
    tid = lax.axis_index("s")
    rbase = tid * RPT
    stsem = (stsem0, stsem1)
    gsem = (gsem0, gsem1)
    ssem = (ssem0, ssem1)

    @pl.when(tid == 0)
    def _():
        pltpu.sync_copy(posx, posx_sh)
        pltpu.sync_copy(posy, posy_sh)

    z16f = jnp.zeros((LANES,), jnp.float32)
    ones16 = jnp.ones((LANES,), jnp.float32)
    zcol = jnp.zeros((LANES,), jnp.int32)

    plsc.subcore_barrier()

    pltpu.sync_copy(posx_sh, posx_v)
    pltpu.sync_copy(posy_sh, posy_v)

    def _zero_acc(acc, zsrc):
        for off in range(0, RPT, CE):
            sz = min(CE, RPT - off)
            src = zsrc if sz == CE else zsrc.at[pl.ds(0, sz)]
            pltpu.sync_copy(src, acc.at[pl.ds(rbase + off, sz)])

    for q in range(NQ):
        def _zrow(r, _):
            for j in range(CH // LANES):
                wxb_v[0, r, pl.ds(j * LANES, LANES)] = z16f
            return _
        lax.fori_loop(0, CE, _zrow, None)
        _zero_acc(sx_acc, wxb_v.at[0])
        _zero_acc(sy_acc, wxb_v.at[0])
        _zero_acc(s1_acc, wxb_v.at[0])
        if q == 0:
            def _zscr(i, _):
                f = lax.iota(jnp.int32, LANES) + i * LANES
                plsc.store_scatter(
                    scr_v, [f // (CE * 8), (f // 8) % CE, f % 8], z16f)
                return _
            lax.fori_loop(0, (2 * CE * 8) // LANES, _zscr, None)
            _zero_acc(scal_acc, scr_v.at[0])

        plsc.subcore_barrier()

        def issue_stage(c, p):
            ebase = (tid * CPT + c) * CE
            pltpu.async_copy(srcp.at[pl.ds(ebase, CE)], src_v.at[p], stsem[p])
            pltpu.async_copy(dstp.at[pl.ds(ebase, CE)], dst_v.at[p], stsem[p])

        def wait_stage(p):
            pltpu.make_async_copy(
                srcp.at[pl.ds(0, CE)], src_v.at[p], stsem[p]).wait()
            pltpu.make_async_copy(
                dstp.at[pl.ds(0, CE)], dst_v.at[p], stsem[p]).wait()

        def header(p):
            pf = jnp.full((LANES,), p, jnp.int32)
            for i in range(CE // LANES):
                sl = pl.ds(i * LANES, LANES)
                sv = src_v[p, sl]
                dv = dst_v[p, sl]
                dvc = jnp.minimum(dv, N - 1)
                pxs = plsc.load_gather(posx_v, [sv])
                pys = plsc.load_gather(posy_v, [sv])
                pxd = plsc.load_gather(posx_v, [dvc])
                pyd = plsc.load_gather(posy_v, [dvc])
                dx = pxd - pxs
                dy = pyd - pys
                sc = 1.0 / (dx * dx + dy * dy + 0.01)
                wx = dx * sc
                wy = dy * sc
                wx_v[p, sl] = wx
                wy_v[p, sl] = wy
                idx2_v[p, sl] = sv * 4 + (half + 2 * q)
                dsti_v[p, sl] = dv
                if q == 0:
                    ev = lax.iota(jnp.int32, LANES) + (i * LANES)
                    plsc.store_scatter(scr_v, [pf, ev, zcol], wx)
                    plsc.store_scatter(scr_v, [pf, ev, zcol + 1], wy)
                    plsc.store_scatter(scr_v, [pf, ev, zcol + 2], ones16)

        def issue_gather(p):
            pltpu.async_copy(x4.at[idx2_v.at[p]], xs_v.at[p], gsem[p])

        def wait_gather(p):
            pltpu.make_async_copy(
                x4.at[pl.ds(0, CE)], xs_v.at[p], gsem[p]).wait()

        def products(p):
            def _egrp(i, _):
                wxvec = wx_v[p, pl.ds(i * LANES, LANES)]
                wyvec = wy_v[p, pl.ds(i * LANES, LANES)]
                for l in range(LANES):
                    e = i * LANES + l
                    wxs = jnp.full((LANES,), wxvec[l], jnp.float32)
                    wys = jnp.full((LANES,), wyvec[l], jnp.float32)
                    for j in range(CH // LANES):
                        csl = pl.ds(j * LANES, LANES)
                        v = xs_v[p, e, csl]
                        wxb_v[p, e, csl] = v * wxs
                        wyb_v[p, e, csl] = v * wys
                return _
            lax.fori_loop(0, CE // LANES, _egrp, None)

        def issue_scatter(p):
            di = dsti_v.at[p]
            pltpu.async_copy(xs_v.at[p], s1_acc.at[di], ssem[p], add=True)
            pltpu.async_copy(wxb_v.at[p], sx_acc.at[di], ssem[p], add=True)
            pltpu.async_copy(wyb_v.at[p], sy_acc.at[di], ssem[p], add=True)
            if q == 0:
                @pl.when(half == 0)
                def _():
                    pltpu.async_copy(scr_v.at[p], scal_acc.at[di], ssem[p],
                                     add=True)

        def wait_scatter(p):
            pltpu.make_async_copy(
                xs_v.at[p], s1_acc.at[pl.ds(0, CE)], ssem[p]).wait()
            pltpu.make_async_copy(
                wxb_v.at[p], sx_acc.at[pl.ds(0, CE)], ssem[p]).wait()
            pltpu.make_async_copy(
                wyb_v.at[p], sy_acc.at[pl.ds(0, CE)], ssem[p]).wait()
            if q == 0:
                @pl.when(half == 0)
                def _():
                    pltpu.make_async_copy(
                        scr_v.at[p], scal_acc.at[pl.ds(0, CE)],
                        ssem[p]).wait()

        NP = CPT // 2

        def step(c, c2, p, pn, first, last):
            wait_gather(p)
            products(p)
            issue_scatter(p)
            if not last:
                wait_stage(pn)
                if first:
                    @pl.when(c2 >= 1)
                    def _():
                        wait_scatter(pn)
                else:
                    wait_scatter(pn)
                header(pn)
                issue_gather(pn)

                @pl.when(c2 < NP - 1)
                def _():
                    issue_stage(c + 2, p)

        issue_stage(0, 0)
        wait_stage(0)
        header(0)
        issue_gather(0)
        issue_stage(1, 1)

        def _pair(c2, _):
            a = c2 * 2
            step(a, c2, 0, 1, True, False)

            @pl.when(c2 < NP - 1)
            def _():
                step(a + 1, c2, 1, 0, False, False)

            @pl.when(c2 == NP - 1)
            def _():
                step(a + 1, c2, 1, 0, False, True)
            return _

        lax.fori_loop(0, NP, _pair, None)

        wait_scatter(0)
        wait_scatter(1)

        plsc.subcore_barrier()

        pltpu.sync_copy(sx_acc.at[pl.ds(rbase, RPT)],
                        sx_out.at[q, half, pl.ds(rbase, RPT)])
        pltpu.sync_copy(sy_acc.at[pl.ds(rbase, RPT)],
                        sy_out.at[q, half, pl.ds(rbase, RPT)])
        pltpu.sync_copy(s1_acc.at[pl.ds(rbase, RPT)],
                        s1_out.at[q, half, pl.ds(rbase, RPT)])
        if q == 0:
            @pl.when(half == 0)
            def _():
                pltpu.sync_copy(scal_acc.at[pl.ds(rbase, RPT)],
                                scal_out.at[pl.ds(rbase, RPT)])


def _edge_aggregate(x4, posx, posy, srcp, dstp):
    mesh = plsc.VectorSubcoreMesh(core_axis_name="c", subcore_axis_name="s",
                                  num_cores=NCORE, num_subcores=NSUB)
    f = pl.kernel(
        _edge_agg_body,
        out_type=(
            jax.ShapeDtypeStruct((NQ, NCORE, NACC, CH), jnp.float32),
            jax.ShapeDtypeStruct((NQ, NCORE, NACC, CH), jnp.float32),
            jax.ShapeDtypeStruct((NQ, NCORE, NACC, CH), jnp.float32),
            jax.ShapeDtypeStruct((NACC, 8), jnp.float32),
        ),
        mesh=mesh,
        compiler_params=pltpu.CompilerParams(
            needs_layout_passes=False, use_tc_tiling_on_sc=False),
        scratch_types=[
            pltpu.VMEM_SHARED((NACC, CH), jnp.float32),
            pltpu.VMEM_SHARED((NACC, CH), jnp.float32),
            pltpu.VMEM_SHARED((NACC, CH), jnp.float32),
            pltpu.VMEM_SHARED((NACC, 8), jnp.float32),
            pltpu.VMEM_SHARED((N,), jnp.float32),
            pltpu.VMEM_SHARED((N,), jnp.float32),
            pltpu.VMEM((N,), jnp.float32),
            pltpu.VMEM((N,), jnp.float32),
            pltpu.VMEM((2, CE), jnp.int32),
            pltpu.VMEM((2, CE), jnp.int32),
            pltpu.VMEM((2, CE), jnp.int32),
            pltpu.VMEM((2, CE), jnp.int32),
            pltpu.VMEM((2, CE), jnp.float32),
            pltpu.VMEM((2, CE), jnp.float32),
            pltpu.VMEM((2, CE, CH), jnp.float32),
            pltpu.VMEM((2, CE, CH), jnp.float32),
            pltpu.VMEM((2, CE, CH), jnp.float32),
            pltpu.VMEM((2, CE, 8), jnp.float32),
            pltpu.SemaphoreType.DMA,
            pltpu.SemaphoreType.DMA,
            pltpu.SemaphoreType.DMA,
            pltpu.SemaphoreType.DMA,
            pltpu.SemaphoreType.DMA,
            pltpu.SemaphoreType.DMA,
        ],
    )
    return f(x4, posx, posy, srcp, dstp)


def _dense_body(x_ref, sx_ref, sy_ref, s1_ref, scal_ref,
                wc_ref, bc_ref, w2_ref, b2_ref, out_ref):
    x = x_ref[...]
    sx = jnp.concatenate(
        [sx_ref[0, 0], sx_ref[0, 1], sx_ref[1, 0], sx_ref[1, 1]], axis=-1)
    sy = jnp.concatenate(
        [sy_ref[0, 0], sy_ref[0, 1], sy_ref[1, 0], sy_ref[1, 1]], axis=-1)
    s1 = jnp.concatenate(
        [s1_ref[0, 0], s1_ref[0, 1], s1_ref[1, 0], s1_ref[1, 1]], axis=-1)
    swx = scal_ref[:, 0:1]
    swy = scal_ref[:, 1:2]
    cnt = scal_ref[:, 2:3]
    rc = 1.0 / jnp.maximum(cnt, 1.0)
    a0 = (x * swx - sx) * rc
    a1 = (x * swy - sy) * rc
    a2 = s1 * rc
    prop = jnp.concatenate([a0, a1, a2, x], axis=-1)
    h = jnp.dot(prop, wc_ref[...], preferred_element_type=jnp.float32)
    h = h + bc_ref[...]
    h = jnp.where(h > 0, h, jnp.exp(h) - 1.0)
    h2 = jnp.dot(h, w2_ref[...], preferred_element_type=jnp.float32)
    h2 = h2 + b2_ref[...]
    out_ref[...] = jnp.where(h2 > 0, h2, jnp.exp(h2) - 1.0)


def _dense(x, sx2, sy2, s12, scal, wc, bc, w2, b2, interpret=False):
    RB = 1000
    grid = (N // RB,)
    return pl.pallas_call(
        _dense_body,
        grid=grid,
        in_specs=[
            pl.BlockSpec((RB, C), lambda i: (i, 0)),
            pl.BlockSpec((NQ, NCORE, RB, CH), lambda i: (0, 0, i, 0)),
            pl.BlockSpec((NQ, NCORE, RB, CH), lambda i: (0, 0, i, 0)),
            pl.BlockSpec((NQ, NCORE, RB, CH), lambda i: (0, 0, i, 0)),
            pl.BlockSpec((RB, 8), lambda i: (i, 0)),
            pl.BlockSpec((4 * C, C), lambda i: (0, 0)),
            pl.BlockSpec((1, C), lambda i: (0, 0)),
            pl.BlockSpec((C, C), lambda i: (0, 0)),
            pl.BlockSpec((1, C), lambda i: (0, 0)),
        ],
        out_specs=pl.BlockSpec((RB, C), lambda i: (i, 0)),
        out_shape=jax.ShapeDtypeStruct((N, C), jnp.float32),
        interpret=interpret,
    )(x, sx2, sy2, s12, scal, wc, bc, w2, b2)


def _insert(nd, ni, d1, i1, d2, i2, d3, i3):
    b1 = nd < d1
    b2 = nd < d2
    b3 = nd < d3
    d3n = jnp.where(b2, d2, jnp.where(b3, nd, d3))
    i3n = jnp.where(b2, i2, jnp.where(b3, ni, i3))
    d2n = jnp.where(b1, d1, jnp.where(b2, nd, d2))
    i2n = jnp.where(b1, i1, jnp.where(b2, ni, i2))
    d1n = jnp.where(b1, nd, d1)
    i1n = jnp.where(b1, ni, i1)
    return d1n, i1n, d2n, i2n, d3n, i3n


def _knn_body(qx_ref, qy_ref, px_ref, py_ref, gidx_ref):
    qx = qx_ref[...]
    qy = qy_ref[...]
    BIG = jnp.float32(3.4e38)
    BIGI = jnp.int32(2 ** 30)
    Dg = jnp.full((QB, NG), BIG)
    for j in range(GS):
        px = px_ref[0:1, pl.ds(j * NG, NG)]
        py = py_ref[0:1, pl.ds(j * NG, NG)]
        ddx = qx - px
        ddy = qy - py
        Dg = jnp.minimum(Dg, ddx * ddx + ddy * ddy)
    cols = lax.broadcasted_iota(jnp.int32, (QB, NG), 1)
    gs = []
    for _k in range(K):
        m = jnp.min(Dg, axis=1, keepdims=True)
        ci = jnp.min(jnp.where(Dg <= m, cols, BIGI), axis=1, keepdims=True)
        gs.append(ci)
        Dg = jnp.where(cols == ci, BIG, Dg)
    gidx_ref[...] = jnp.concatenate(
        gs + [jnp.zeros((QB, 1), jnp.int32)], axis=-1)


def _knn(qx, qy, pxp, pyp, interpret=False):
    grid = (M // QB,)
    return pl.pallas_call(
        _knn_body,
        grid=grid,
        in_specs=[
            pl.BlockSpec((QB, 1), lambda i: (i, 0)),
            pl.BlockSpec((QB, 1), lambda i: (i, 0)),
            pl.BlockSpec((1, N), lambda i: (0, 0)),
            pl.BlockSpec((1, N), lambda i: (0, 0)),
        ],
        out_specs=pl.BlockSpec((QB, 4), lambda i: (i, 0)),
        out_shape=jax.ShapeDtypeStruct((M, 4), jnp.int32),
        interpret=interpret,
    )(qx, qy, pxp, pyp)


def _lexlt(da, ca, db, cb):
    return (da < db) | ((da == db) & (ca < cb))


def _interp_body(h_hbm, gsel_hbm, qp_hbm, posx, posy, y_hbm,
                 posx_sh, posy_sh, posx_v, posy_v,
                 gsel_v, qp_v, idx_v, w_v, rows_v, out_v, sem):
    half = lax.axis_index("c")
    tid = lax.axis_index("s")
    wid = tid * NCORE + half

    @pl.when(tid == 0)
    def _():
        pltpu.sync_copy(posx, posx_sh)
        pltpu.sync_copy(posy, posy_sh)

    plsc.subcore_barrier()
    pltpu.sync_copy(posx_sh, posx_v)
    pltpu.sync_copy(posy_sh, posy_v)

    BIG = jnp.float32(3.4e38)
    i16 = lax.iota(jnp.int32, LANES)
    msk8 = i16 < 8
    msk3 = i16 < 3
    j8 = i16 % 8

    def _chunk(c, _):
        g = wid * GPT + c
        pltpu.sync_copy(gsel_hbm.at[g], gsel_v)
        pltpu.sync_copy(qp_hbm.at[g], qp_v)

        def _sel(qg, _):
            sl = pl.ds(qg * LANES, LANES)
            g1v = gsel_v[0, sl]
            g2v = gsel_v[1, sl]
            g3v = gsel_v[2, sl]
            qxv = qp_v[0, sl]
            qyv = qp_v[1, sl]
            for l in range(LANES):
                pt = qg * LANES + l
                qxs = jnp.full((LANES,), qxv[l], jnp.float32)
                qys = jnp.full((LANES,), qyv[l], jnp.float32)
                ga = jnp.full((LANES,), g1v[l], jnp.int32)
                gb = jnp.full((LANES,), g2v[l], jnp.int32)
                gc = jnp.full((LANES,), g3v[l], jnp.int32)
                c12 = jnp.where(msk8, ga, gb) * GS + j8
                c3 = gc * GS + j8
                pxa = plsc.load_gather(posx_v, [c12])
                pya = plsc.load_gather(posy_v, [c12])
                pxb = plsc.load_gather(posx_v, [c3])
                pyb = plsc.load_gather(posy_v, [c3])
                dxa = qxs - pxa
                dya = qys - pya
                d12 = dxa * dxa + dya * dya
                dxb = qxs - pxb
                dyb = qys - pyb
                d3 = dxb * dxb + dyb * dyb
                d3 = jnp.where(msk8, d3, BIG)
                sk1, sv1 = plsc.sort_key_val(d12, c12)
                sk2, sv2 = plsc.sort_key_val(d3, c3)
                m1, m2, m3 = sk1[0], sk1[1], sk1[2]
                n1, n2, n3 = sv1[0], sv1[1], sv1[2]
                for (dv, cv) in ((sk2[0], sv2[0]), (sk2[1], sv2[1]),
                                 (sk2[2], sv2[2])):
                    b1 = _lexlt(dv, cv, m1, n1)
                    b2 = _lexlt(dv, cv, m2, n2)
                    b3 = _lexlt(dv, cv, m3, n3)
                    m3n = jnp.where(b2, m2, jnp.where(b3, dv, m3))
                    n3n = jnp.where(b2, n2, jnp.where(b3, cv, n3))
                    m2n = jnp.where(b1, m1, jnp.where(b2, dv, m2))
                    n2n = jnp.where(b1, n1, jnp.where(b2, cv, n2))
                    m1 = jnp.where(b1, dv, m1)
                    n1 = jnp.where(b1, cv, n1)
                    m2, n2, m3, n3 = m2n, n2n, m3n, n3n
                dsel = jnp.where(i16 == 0, jnp.full((LANES,), m1),
                                 jnp.where(i16 == 1, jnp.full((LANES,), m2),
                                           jnp.full((LANES,), m3)))
                csel = jnp.where(i16 == 0, jnp.full((LANES,), n1),
                                 jnp.where(i16 == 1, jnp.full((LANES,), n2),
                                           jnp.full((LANES,), n3)))
                w = 1.0 / jnp.maximum(dsel, 1e-16)
                svec = (jnp.full((LANES,), w[0]) + jnp.full((LANES,), w[1])
                        + jnp.full((LANES,), w[2]))
                wn = w / svec
                pfull = jnp.full((LANES,), pt, jnp.int32)
                plsc.store_scatter(w_v, [i16, pfull], wn, mask=msk3)
                plsc.store_scatter(idx_v, [i16, pfull], csel, mask=msk3)
            return _
        lax.fori_loop(0, PC // LANES, _sel, None)

        cps = []
        for a in range(K):
            cps.append(pltpu.async_copy(
                h_hbm.at[idx_v.at[a]], rows_v.at[pl.ds(a * PC, PC)], sem))
        for cp in cps:
            cp.wait()

        def _pt(q, _):
            wv0 = w_v[0, pl.ds(q * LANES, LANES)]
            wv1 = w_v[1, pl.ds(q * LANES, LANES)]
            wv2 = w_v[2, pl.ds(q * LANES, LANES)]
            for l in range(LANES):
                pt = q * LANES + l
                w0 = jnp.full((LANES,), wv0[l], jnp.float32)
                w1 = jnp.full((LANES,), wv1[l], jnp.float32)
                w2 = jnp.full((LANES,), wv2[l], jnp.float32)
                for j in range(C // LANES):
                    csl = pl.ds(j * LANES, LANES)
                    acc = rows_v[pt, csl] * w0
                    acc = acc + rows_v[PC + pt, csl] * w1
                    acc = acc + rows_v[2 * PC + pt, csl] * w2
                    out_v[pt, csl] = acc
            return _
        lax.fori_loop(0, PC // LANES, _pt, None)
        pltpu.sync_copy(out_v, y_hbm.at[pl.ds(g * PC, PC)])
        return _

    lax.fori_loop(0, GPT, _chunk, None)


def _interpolate(h, gsel, qp, posx, posy):
    mesh = plsc.VectorSubcoreMesh(core_axis_name="c", subcore_axis_name="s",
                                  num_cores=NCORE, num_subcores=NSUB)
    f = pl.kernel(
        _interp_body,
        out_type=jax.ShapeDtypeStruct((MPAD, C), jnp.float32),
        mesh=mesh,
        compiler_params=pltpu.CompilerParams(needs_layout_passes=False),
        scratch_types=[
            pltpu.VMEM_SHARED((N,), jnp.float32),
            pltpu.VMEM_SHARED((N,), jnp.float32),
            pltpu.VMEM((N,), jnp.float32),
            pltpu.VMEM((N,), jnp.float32),
            pltpu.VMEM((K, PC), jnp.int32),
            pltpu.VMEM((2, PC), jnp.float32),
            pltpu.VMEM((K, PC), jnp.int32),
            pltpu.VMEM((K, PC), jnp.float32),
            pltpu.VMEM((K * PC, C), jnp.float32),
            pltpu.VMEM((PC, C), jnp.float32),
            pltpu.SemaphoreType.DMA,
        ],
    )
    return f(h, gsel, qp, posx, posy)


def kernel(x, pos, pos_skip, W_conv, b_conv, W2, b2,
           edge_index, batch, batch_skip):
    del batch, batch_skip
    x4 = x.reshape(N, 4, CH).reshape(4 * N, CH)
    posx = pos[:, 0]
    posy = pos[:, 1]
    npad = EPAD - E
    srcp = jnp.concatenate(
        [edge_index[0], jnp.zeros((npad,), jnp.int32)])
    dstp = jnp.concatenate(
        [edge_index[1], jnp.full((npad,), N, jnp.int32)])

    sx2, sy2, s12, scal = _edge_aggregate(x4, posx, posy, srcp, dstp)

    h = _dense(x, sx2, sy2, s12, scal,
               W_conv, b_conv.reshape(1, C), W2, b2.reshape(1, C))

    qx = pos_skip[:, 0:1]
    qy = pos_skip[:, 1:2]
    pxp = posx.reshape(NG, GS).T.reshape(1, N)
    pyp = posy.reshape(NG, GS).T.reshape(1, N)
    gidx4 = _knn(qx, qy, pxp, pyp)

    mp = MPAD - M
    gsel = jnp.concatenate(
        [gidx4[:, :K], jnp.zeros((mp, K), jnp.int32)]).reshape(
            MPAD // PC, PC, K).transpose(0, 2, 1)
    qp = jnp.concatenate(
        [pos_skip, jnp.zeros((mp, 2), jnp.float32)]).reshape(
            MPAD // PC, PC, 2).transpose(0, 2, 1)

    y = _interpolate(h, gsel, qp, posx, posy)
    return y[:M]


# speedup vs baseline: 1.0010x; 1.0010x over previous
import jax
import jax.numpy as jnp
from jax import lax
from jax.experimental import pallas as pl
from jax.experimental.pallas import tpu as pltpu
from jax.experimental.pallas import tpu_sc as plsc

N = 10000
E = 160000
C = 128
M = 20000
K = 3

NCORE = 2
NSUB = 16
LANES = 16

CH = C // 4
NQ = 2

CE = 128
CPT = 80
EPAD = CE * CPT * NSUB
NACC = 10240
RPT = NACC // NSUB

QB = 400
GS = 8
NG = N // GS

PC = 128
GPT = 5
MPAD = PC * GPT * NCORE * NSUB


def _edge_agg_body(x4, posx, posy, srcp, dstp,
                   sx_out, sy_out, s1_out, scal_out,
                   sx_acc, sy_acc, s1_acc, scal_acc, posx_sh, posy_sh,
                   posx_v, posy_v, src_v, dst_v, idx2_v, dsti_v,
                   wx_v, wy_v, xs_v, wxb_v, wyb_v, scr_v,
                   stsem0, stsem1, gsem0, gsem1, ssem0, ssem1):
    half = lax.axis_index("c")
    tid = lax.axis_index("s")
    rbase = tid * RPT
    stsem = (stsem0, stsem1)
    gsem = (gsem0, gsem1)
    ssem = (ssem0, ssem1)

    @pl.when(tid == 0)
    def _():
        pltpu.sync_copy(posx, posx_sh)
        pltpu.sync_copy(posy, posy_sh)

    z16f = jnp.zeros((LANES,), jnp.float32)
    ones16 = jnp.ones((LANES,), jnp.float32)
    zcol = jnp.zeros((LANES,), jnp.int32)

    plsc.subcore_barrier()

    pltpu.sync_copy(posx_sh, posx_v)
    pltpu.sync_copy(posy_sh, posy_v)

    def _zero_acc(acc, zsrc):
        for off in range(0, RPT, CE):
            sz = min(CE, RPT - off)
            src = zsrc if sz == CE else zsrc.at[pl.ds(0, sz)]
            pltpu.sync_copy(src, acc.at[pl.ds(rbase + off, sz)])

    for q in range(NQ):
        def _zrow(r, _):
            for j in range(CH // LANES):
                wxb_v[0, r, pl.ds(j * LANES, LANES)] = z16f
            return _
        lax.fori_loop(0, CE, _zrow, None)
        _zero_acc(sx_acc, wxb_v.at[0])
        _zero_acc(sy_acc, wxb_v.at[0])
        _zero_acc(s1_acc, wxb_v.at[0])
        if q == 0:
            def _zscr(i, _):
                f = lax.iota(jnp.int32, LANES) + i * LANES
                plsc.store_scatter(
                    scr_v, [f // (CE * 8), (f // 8) % CE, f % 8], z16f)
                return _
            lax.fori_loop(0, (2 * CE * 8) // LANES, _zscr, None)
            _zero_acc(scal_acc, scr_v.at[0])

        plsc.subcore_barrier()

        def issue_stage(c, p):
            ebase = (tid * CPT + c) * CE
            pltpu.async_copy(srcp.at[pl.ds(ebase, CE)], src_v.at[p], stsem[p])
            pltpu.async_copy(dstp.at[pl.ds(ebase, CE)], dst_v.at[p], stsem[p])

        def wait_stage(p):
            pltpu.make_async_copy(
                srcp.at[pl.ds(0, CE)], src_v.at[p], stsem[p]).wait()
            pltpu.make_async_copy(
                dstp.at[pl.ds(0, CE)], dst_v.at[p], stsem[p]).wait()

        def header(p):
            pf = jnp.full((LANES,), p, jnp.int32)
            for i in range(CE // LANES):
                sl = pl.ds(i * LANES, LANES)
                sv = src_v[p, sl]
                dv = dst_v[p, sl]
                dvc = jnp.minimum(dv, N - 1)
                pxs = plsc.load_gather(posx_v, [sv])
                pys = plsc.load_gather(posy_v, [sv])
                pxd = plsc.load_gather(posx_v, [dvc])
                pyd = plsc.load_gather(posy_v, [dvc])
                dx = pxd - pxs
                dy = pyd - pys
                sc = 1.0 / (dx * dx + dy * dy + 0.01)
                wx = dx * sc
                wy = dy * sc
                wx_v[p, sl] = wx
                wy_v[p, sl] = wy
                idx2_v[p, sl] = sv * 4 + (half + 2 * q)
                dsti_v[p, sl] = dv
                if q == 0:
                    ev = lax.iota(jnp.int32, LANES) + (i * LANES)
                    plsc.store_scatter(scr_v, [pf, ev, zcol], wx)
                    plsc.store_scatter(scr_v, [pf, ev, zcol + 1], wy)
                    plsc.store_scatter(scr_v, [pf, ev, zcol + 2], ones16)

        def issue_gather(p):
            pltpu.async_copy(x4.at[idx2_v.at[p]], xs_v.at[p], gsem[p])

        def wait_gather(p):
            pltpu.make_async_copy(
                x4.at[pl.ds(0, CE)], xs_v.at[p], gsem[p]).wait()

        def products(p):
            def _egrp(i, _):
                wxvec = wx_v[p, pl.ds(i * LANES, LANES)]
                wyvec = wy_v[p, pl.ds(i * LANES, LANES)]
                for l in range(LANES):
                    e = i * LANES + l
                    wxs = jnp.full((LANES,), wxvec[l], jnp.float32)
                    wys = jnp.full((LANES,), wyvec[l], jnp.float32)
                    for j in range(CH // LANES):
                        csl = pl.ds(j * LANES, LANES)
                        v = xs_v[p, e, csl]
                        wxb_v[p, e, csl] = v * wxs
                        wyb_v[p, e, csl] = v * wys
                return _
            lax.fori_loop(0, CE // LANES, _egrp, None)

        def issue_scatter(p):
            di = dsti_v.at[p]
            pltpu.async_copy(xs_v.at[p], s1_acc.at[di], ssem[p], add=True)
            pltpu.async_copy(wxb_v.at[p], sx_acc.at[di], ssem[p], add=True)
            pltpu.async_copy(wyb_v.at[p], sy_acc.at[di], ssem[p], add=True)
            if q == 0:
                @pl.when(half == 0)
                def _():
                    pltpu.async_copy(scr_v.at[p], scal_acc.at[di], ssem[p],
                                     add=True)

        def wait_scatter(p):
            pltpu.make_async_copy(
                xs_v.at[p], s1_acc.at[pl.ds(0, CE)], ssem[p]).wait()
            pltpu.make_async_copy(
                wxb_v.at[p], sx_acc.at[pl.ds(0, CE)], ssem[p]).wait()
            pltpu.make_async_copy(
                wyb_v.at[p], sy_acc.at[pl.ds(0, CE)], ssem[p]).wait()
            if q == 0:
                @pl.when(half == 0)
                def _():
                    pltpu.make_async_copy(
                        scr_v.at[p], scal_acc.at[pl.ds(0, CE)],
                        ssem[p]).wait()

        NP = CPT // 2

        def step(c, c2, p, pn, first, last):
            wait_gather(p)
            products(p)
            issue_scatter(p)
            if not last:
                wait_stage(pn)
                if first:
                    @pl.when(c2 >= 1)
                    def _():
                        wait_scatter(pn)
                else:
                    wait_scatter(pn)
                header(pn)
                issue_gather(pn)

                @pl.when(c2 < NP - 1)
                def _():
                    issue_stage(c + 2, p)

        issue_stage(0, 0)
        wait_stage(0)
        header(0)
        issue_gather(0)
        issue_stage(1, 1)

        def _pair(c2, _):
            a = c2 * 2
            step(a, c2, 0, 1, True, False)

            @pl.when(c2 < NP - 1)
            def _():
                step(a + 1, c2, 1, 0, False, False)

            @pl.when(c2 == NP - 1)
            def _():
                step(a + 1, c2, 1, 0, False, True)
            return _

        lax.fori_loop(0, NP, _pair, None)

        wait_scatter(0)
        wait_scatter(1)

        plsc.subcore_barrier()

        pltpu.sync_copy(sx_acc.at[pl.ds(rbase, RPT)],
                        sx_out.at[q, half, pl.ds(rbase, RPT)])
        pltpu.sync_copy(sy_acc.at[pl.ds(rbase, RPT)],
                        sy_out.at[q, half, pl.ds(rbase, RPT)])
        pltpu.sync_copy(s1_acc.at[pl.ds(rbase, RPT)],
                        s1_out.at[q, half, pl.ds(rbase, RPT)])
        if q == 0:
            @pl.when(half == 0)
            def _():
                pltpu.sync_copy(scal_acc.at[pl.ds(rbase, RPT)],
                                scal_out.at[pl.ds(rbase, RPT)])


def _edge_aggregate(x4, posx, posy, srcp, dstp):
    mesh = plsc.VectorSubcoreMesh(core_axis_name="c", subcore_axis_name="s",
                                  num_cores=NCORE, num_subcores=NSUB)
    f = pl.kernel(
        _edge_agg_body,
        out_type=(
            jax.ShapeDtypeStruct((NQ, NCORE, NACC, CH), jnp.float32),
            jax.ShapeDtypeStruct((NQ, NCORE, NACC, CH), jnp.float32),
            jax.ShapeDtypeStruct((NQ, NCORE, NACC, CH), jnp.float32),
            jax.ShapeDtypeStruct((NACC, 8), jnp.float32),
        ),
        mesh=mesh,
        compiler_params=pltpu.CompilerParams(
            needs_layout_passes=False, use_tc_tiling_on_sc=False),
        scratch_types=[
            pltpu.VMEM_SHARED((NACC, CH), jnp.float32),
            pltpu.VMEM_SHARED((NACC, CH), jnp.float32),
            pltpu.VMEM_SHARED((NACC, CH), jnp.float32),
            pltpu.VMEM_SHARED((NACC, 8), jnp.float32),
            pltpu.VMEM_SHARED((N,), jnp.float32),
            pltpu.VMEM_SHARED((N,), jnp.float32),
            pltpu.VMEM((N,), jnp.float32),
            pltpu.VMEM((N,), jnp.float32),
            pltpu.VMEM((2, CE), jnp.int32),
            pltpu.VMEM((2, CE), jnp.int32),
            pltpu.VMEM((2, CE), jnp.int32),
            pltpu.VMEM((2, CE), jnp.int32),
            pltpu.VMEM((2, CE), jnp.float32),
            pltpu.VMEM((2, CE), jnp.float32),
            pltpu.VMEM((2, CE, CH), jnp.float32),
            pltpu.VMEM((2, CE, CH), jnp.float32),
            pltpu.VMEM((2, CE, CH), jnp.float32),
            pltpu.VMEM((2, CE, 8), jnp.float32),
            pltpu.SemaphoreType.DMA,
            pltpu.SemaphoreType.DMA,
            pltpu.SemaphoreType.DMA,
            pltpu.SemaphoreType.DMA,
            pltpu.SemaphoreType.DMA,
            pltpu.SemaphoreType.DMA,
        ],
    )
    return f(x4, posx, posy, srcp, dstp)


def _dense_body(x_ref, sx_ref, sy_ref, s1_ref, scal_ref,
                wc_ref, bc_ref, w2_ref, b2_ref, out_ref):
    x = x_ref[...]
    sx = jnp.concatenate(
        [sx_ref[0, 0], sx_ref[0, 1], sx_ref[1, 0], sx_ref[1, 1]], axis=-1)
    sy = jnp.concatenate(
        [sy_ref[0, 0], sy_ref[0, 1], sy_ref[1, 0], sy_ref[1, 1]], axis=-1)
    s1 = jnp.concatenate(
        [s1_ref[0, 0], s1_ref[0, 1], s1_ref[1, 0], s1_ref[1, 1]], axis=-1)
    swx = scal_ref[:, 0:1]
    swy = scal_ref[:, 1:2]
    cnt = scal_ref[:, 2:3]
    rc = 1.0 / jnp.maximum(cnt, 1.0)
    a0 = (x * swx - sx) * rc
    a1 = (x * swy - sy) * rc
    a2 = s1 * rc
    prop = jnp.concatenate([a0, a1, a2, x], axis=-1)
    h = jnp.dot(prop, wc_ref[...], preferred_element_type=jnp.float32)
    h = h + bc_ref[...]
    h = jnp.where(h > 0, h, jnp.exp(h) - 1.0)
    h2 = jnp.dot(h, w2_ref[...], preferred_element_type=jnp.float32)
    h2 = h2 + b2_ref[...]
    out_ref[...] = jnp.where(h2 > 0, h2, jnp.exp(h2) - 1.0)


def _dense(x, sx2, sy2, s12, scal, wc, bc, w2, b2, interpret=False):
    RB = 1000
    grid = (N // RB,)
    return pl.pallas_call(
        _dense_body,
        grid=grid,
        in_specs=[
            pl.BlockSpec((RB, C), lambda i: (i, 0)),
            pl.BlockSpec((NQ, NCORE, RB, CH), lambda i: (0, 0, i, 0)),
            pl.BlockSpec((NQ, NCORE, RB, CH), lambda i: (0, 0, i, 0)),
            pl.BlockSpec((NQ, NCORE, RB, CH), lambda i: (0, 0, i, 0)),
            pl.BlockSpec((RB, 8), lambda i: (i, 0)),
            pl.BlockSpec((4 * C, C), lambda i: (0, 0)),
            pl.BlockSpec((1, C), lambda i: (0, 0)),
            pl.BlockSpec((C, C), lambda i: (0, 0)),
            pl.BlockSpec((1, C), lambda i: (0, 0)),
        ],
        out_specs=pl.BlockSpec((RB, C), lambda i: (i, 0)),
        out_shape=jax.ShapeDtypeStruct((N, C), jnp.float32),
        interpret=interpret,
    )(x, sx2, sy2, s12, scal, wc, bc, w2, b2)


def _knn_body(qx_ref, qy_ref, px_ref, py_ref, gidx_ref):
    qx = qx_ref[...]
    qy = qy_ref[...]
    BIG = jnp.float32(3.4e38)
    BIGI = jnp.int32(2 ** 30)
    Dg = jnp.full((QB, NG), BIG)
    for j in range(GS):
        px = px_ref[0:1, pl.ds(j * NG, NG)]
        py = py_ref[0:1, pl.ds(j * NG, NG)]
        ddx = qx - px
        ddy = qy - py
        Dg = jnp.minimum(Dg, ddx * ddx + ddy * ddy)
    cols = lax.broadcasted_iota(jnp.int32, (QB, NG), 1)
    gs = []
    for _k in range(K):
        m = jnp.min(Dg, axis=1, keepdims=True)
        ci = jnp.min(jnp.where(Dg <= m, cols, BIGI), axis=1, keepdims=True)
        gs.append(ci)
        Dg = jnp.where(cols == ci, BIG, Dg)
    gidx_ref[...] = jnp.concatenate(
        gs + [jnp.zeros((QB, 1), jnp.int32)], axis=-1)


def _knn(qx, qy, pxp, pyp, interpret=False):
    grid = (M // QB,)
    return pl.pallas_call(
        _knn_body,
        grid=grid,
        in_specs=[
            pl.BlockSpec((QB, 1), lambda i: (i, 0)),
            pl.BlockSpec((QB, 1), lambda i: (i, 0)),
            pl.BlockSpec((1, N), lambda i: (0, 0)),
            pl.BlockSpec((1, N), lambda i: (0, 0)),
        ],
        out_specs=pl.BlockSpec((QB, 4), lambda i: (i, 0)),
        out_shape=jax.ShapeDtypeStruct((M, 4), jnp.int32),
        interpret=interpret,
    )(qx, qy, pxp, pyp)


def _lexlt(da, ca, db, cb):
    return (da < db) | ((da == db) & (ca < cb))


def _interp_body(h_hbm, gsel_hbm, qp_hbm, posx, posy, y_hbm,
                 posx_sh, posy_sh, posx_v, posy_v,
                 gsel_v, qp_v, idx_v, w_v, rows_v, out_v, sem):
    half = lax.axis_index("c")
    tid = lax.axis_index("s")
    wid = tid * NCORE + half

    @pl.when(tid == 0)
    def _():
        pltpu.sync_copy(posx, posx_sh)
        pltpu.sync_copy(posy, posy_sh)

    plsc.subcore_barrier()
    pltpu.sync_copy(posx_sh, posx_v)
    pltpu.sync_copy(posy_sh, posy_v)

    BIG = jnp.float32(3.4e38)
    i16 = lax.iota(jnp.int32, LANES)
    msk8 = i16 < 8
    msk3 = i16 < 3
    j8 = i16 % 8

    def _chunk(c, _):
        g = wid * GPT + c
        pltpu.sync_copy(gsel_hbm.at[g], gsel_v)
        pltpu.sync_copy(qp_hbm.at[g], qp_v)

        def _sel(qg, _):
            sl = pl.ds(qg * LANES, LANES)
            g1v = gsel_v[0, sl]
            g2v = gsel_v[1, sl]
            g3v = gsel_v[2, sl]
            qxv = qp_v[0, sl]
            qyv = qp_v[1, sl]
            for l in range(LANES):
                pt = qg * LANES + l
                qxs = jnp.full((LANES,), qxv[l], jnp.float32)
                qys = jnp.full((LANES,), qyv[l], jnp.float32)
                ga = jnp.full((LANES,), g1v[l], jnp.int32)
                gb = jnp.full((LANES,), g2v[l], jnp.int32)
                gc = jnp.full((LANES,), g3v[l], jnp.int32)
                c12 = jnp.where(msk8, ga, gb) * GS + j8
                c3 = gc * GS + j8
                pxa = plsc.load_gather(posx_v, [c12])
                pya = plsc.load_gather(posy_v, [c12])
                pxb = plsc.load_gather(posx_v, [c3])
                pyb = plsc.load_gather(posy_v, [c3])
                dxa = qxs - pxa
                dya = qys - pya
                d12 = dxa * dxa + dya * dya
                dxb = qxs - pxb
                dyb = qys - pyb
                d3 = dxb * dxb + dyb * dyb
                d3 = jnp.where(msk8, d3, BIG)
                sk1, sv1 = plsc.sort_key_val(d12, c12)
                sk2, sv2 = plsc.sort_key_val(d3, c3)
                m1, m2, m3 = sk1[0], sk1[1], sk1[2]
                n1, n2, n3 = sv1[0], sv1[1], sv1[2]
                for (dv, cv) in ((sk2[0], sv2[0]), (sk2[1], sv2[1]),
                                 (sk2[2], sv2[2])):
                    b1 = _lexlt(dv, cv, m1, n1)
                    b2 = _lexlt(dv, cv, m2, n2)
                    b3 = _lexlt(dv, cv, m3, n3)
                    m3n = jnp.where(b2, m2, jnp.where(b3, dv, m3))
                    n3n = jnp.where(b2, n2, jnp.where(b3, cv, n3))
                    m2n = jnp.where(b1, m1, jnp.where(b2, dv, m2))
                    n2n = jnp.where(b1, n1, jnp.where(b2, cv, n2))
                    m1 = jnp.where(b1, dv, m1)
                    n1 = jnp.where(b1, cv, n1)
                    m2, n2, m3, n3 = m2n, n2n, m3n, n3n
                dsel = jnp.where(i16 == 0, jnp.full((LANES,), m1),
                                 jnp.where(i16 == 1, jnp.full((LANES,), m2),
                                           jnp.full((LANES,), m3)))
                csel = jnp.where(i16 == 0, jnp.full((LANES,), n1),
                                 jnp.where(i16 == 1, jnp.full((LANES,), n2),
                                           jnp.full((LANES,), n3)))
                w = 1.0 / jnp.maximum(dsel, 1e-16)
                svec = (jnp.full((LANES,), w[0]) + jnp.full((LANES,), w[1])
                        + jnp.full((LANES,), w[2]))
                wn = w / svec
                pfull = jnp.full((LANES,), pt, jnp.int32)
                plsc.store_scatter(w_v, [i16, pfull], wn, mask=msk3)
                plsc.store_scatter(idx_v, [i16, pfull], csel, mask=msk3)
            return _
        lax.fori_loop(0, PC // LANES, _sel, None)

        cps = []
        for a in range(K):
            cps.append(pltpu.async_copy(
                h_hbm.at[idx_v.at[a]], rows_v.at[pl.ds(a * PC, PC)], sem))
        for cp in cps:
            cp.wait()

        def _pt(q, _):
            wv0 = w_v[0, pl.ds(q * LANES, LANES)]
            wv1 = w_v[1, pl.ds(q * LANES, LANES)]
            wv2 = w_v[2, pl.ds(q * LANES, LANES)]
            for l in range(LANES):
                pt = q * LANES + l
                w0 = jnp.full((LANES,), wv0[l], jnp.float32)
                w1 = jnp.full((LANES,), wv1[l], jnp.float32)
                w2 = jnp.full((LANES,), wv2[l], jnp.float32)
                for j in range(C // LANES):
                    csl = pl.ds(j * LANES, LANES)
                    acc = rows_v[pt, csl] * w0
                    acc = acc + rows_v[PC + pt, csl] * w1
                    acc = acc + rows_v[2 * PC + pt, csl] * w2
                    out_v[pt, csl] = acc
            return _
        lax.fori_loop(0, PC // LANES, _pt, None)
        pltpu.sync_copy(out_v, y_hbm.at[pl.ds(g * PC, PC)])
        return _

    lax.fori_loop(0, GPT, _chunk, None)


def _interpolate(h, gsel, qp, posx, posy):
    mesh = plsc.VectorSubcoreMesh(core_axis_name="c", subcore_axis_name="s",
                                  num_cores=NCORE, num_subcores=NSUB)
    f = pl.kernel(
        _interp_body,
        out_type=jax.ShapeDtypeStruct((MPAD, C), jnp.float32),
        mesh=mesh,
        compiler_params=pltpu.CompilerParams(needs_layout_passes=False),
        scratch_types=[
            pltpu.VMEM_SHARED((N,), jnp.float32),
            pltpu.VMEM_SHARED((N,), jnp.float32),
            pltpu.VMEM((N,), jnp.float32),
            pltpu.VMEM((N,), jnp.float32),
            pltpu.VMEM((K, PC), jnp.int32),
            pltpu.VMEM((2, PC), jnp.float32),
            pltpu.VMEM((K, PC), jnp.int32),
            pltpu.VMEM((K, PC), jnp.float32),
            pltpu.VMEM((K * PC, C), jnp.float32),
            pltpu.VMEM((PC, C), jnp.float32),
            pltpu.SemaphoreType.DMA,
        ],
    )
    return f(h, gsel, qp, posx, posy)


def kernel(x, pos, pos_skip, W_conv, b_conv, W2, b2,
           edge_index, batch, batch_skip):
    del batch, batch_skip
    x4 = x.reshape(N, 4, CH).reshape(4 * N, CH)
    posx = pos[:, 0]
    posy = pos[:, 1]
    npad = EPAD - E
    srcp = jnp.concatenate(
        [edge_index[0], jnp.zeros((npad,), jnp.int32)])
    dstp = jnp.concatenate(
        [edge_index[1], jnp.full((npad,), N, jnp.int32)])

    sx2, sy2, s12, scal = _edge_aggregate(x4, posx, posy, srcp, dstp)

    h = _dense(x, sx2, sy2, s12, scal,
               W_conv, b_conv.reshape(1, C), W2, b2.reshape(1, C))

    qx = pos_skip[:, 0:1]
    qy = pos_skip[:, 1:2]
    pxp = posx.reshape(NG, GS).T.reshape(1, N)
    pyp = posy.reshape(NG, GS).T.reshape(1, N)
    gidx4 = _knn(qx, qy, pxp, pyp)

    mp = MPAD - M
    gsel = jnp.concatenate(
        [gidx4[:, :K], jnp.zeros((mp, K), jnp.int32)]).reshape(
            MPAD // PC, PC, K).transpose(0, 2, 1)
    qp = jnp.concatenate(
        [pos_skip, jnp.zeros((mp, 2), jnp.float32)]).reshape(
            MPAD // PC, PC, 2).transpose(0, 2, 1)

    y = _interpolate(h, gsel, qp, posx, posy)
    return y[:M]
